# baseline (speedup 1.0000x reference)
import functools

import jax
import jax.numpy as jnp
from jax import lax
from jax.experimental import pallas as pl
from jax.experimental.pallas import tpu as pltpu
from jax.experimental.pallas import tpu_sc as plsc

VALUE_MAX = 300.0
NUM_BINS = 601
LANES = 16
NUM_WORKERS = 32


def _two_hot(s):
    pos = jnp.clip(s, -VALUE_MAX, VALUE_MAX) + VALUE_MAX
    li = pos.astype(jnp.int32)
    uw = pos - li.astype(jnp.float32)
    lw = 1.0 - uw
    ui = jnp.minimum(li + 1, NUM_BINS - 1)
    return li, ui, lw, uw


def _make_sc_kernel(batch):
    cols_per_worker = batch // NUM_WORKERS
    chunk_cols = 128
    n_chunks = cols_per_worker // chunk_cols
    groups = chunk_cols // LANES

    mesh = plsc.VectorSubcoreMesh(core_axis_name="c", subcore_axis_name="s")

    @functools.partial(
        pl.kernel,
        out_type=jax.ShapeDtypeStruct((NUM_BINS, batch), jnp.float32),
        mesh=mesh,
        scratch_types=[
            pltpu.VMEM((cols_per_worker,), jnp.float32),
            pltpu.VMEM((NUM_BINS, chunk_cols), jnp.float32),
        ],
        compiler_params=pltpu.CompilerParams(needs_layout_passes=False),
    )
    def body(scalar_hbm, out_hbm, scal_v, buf):
        wid = lax.axis_index("c") * 16 + lax.axis_index("s")
        col0 = wid * cols_per_worker

        pltpu.sync_copy(scalar_hbm.at[pl.ds(col0, cols_per_worker)], scal_v)

        zeros16 = jnp.zeros((LANES,), jnp.float32)
        lane = lax.iota(jnp.int32, LANES)

        def zbody(r, carry):
            for k in range(groups):
                buf[r, pl.ds(k * LANES, LANES)] = zeros16
            return carry

        lax.fori_loop(0, NUM_BINS, zbody, 0)

        def chunk_body(c, carry):
            for g in range(groups):
                s = scal_v[pl.ds(c * chunk_cols + g * LANES, LANES)]
                li, ui, lw, uw = _two_hot(s)
                cols = lane + g * LANES
                plsc.store_scatter(buf, [ui, cols], uw)
                plsc.store_scatter(buf, [li, cols], lw)
            pltpu.sync_copy(
                buf, out_hbm.at[:, pl.ds(col0 + c * chunk_cols, chunk_cols)])
            for g in range(groups):
                s = scal_v[pl.ds(c * chunk_cols + g * LANES, LANES)]
                li, ui, _, _ = _two_hot(s)
                cols = lane + g * LANES
                plsc.store_scatter(buf, [ui, cols], zeros16)
                plsc.store_scatter(buf, [li, cols], zeros16)
            return carry

        lax.fori_loop(0, n_chunks, chunk_body, 0)

    return body


def kernel(scalar):
    out_t = _make_sc_kernel(scalar.shape[0])(scalar)
    return out_t.T
